# Initial kernel scaffold; baseline (speedup 1.0000x reference)
import functools

import jax
import jax.numpy as jnp
from jax import lax
from jax.experimental import pallas as pl
from jax.experimental.pallas import tpu as pltpu
from jax.experimental.pallas import tpu_sc as plsc

NC = 2
NS = 16
NW = NC * NS
LANES = 16

_CHUNK = 1024
_GB = 128


def _rsqrt(x):
    xi = plsc.bitcast(x, jnp.int32)
    y = plsc.bitcast(jnp.int32(0x5F3759DF) - (xi >> 1), jnp.float32)
    for _ in range(3):
        y = y * (jnp.float32(1.5) - jnp.float32(0.5) * x * y * y)
    return y


def _full(v):
    return jnp.full((LANES,), v, jnp.int32)


def _build_tables(pos_pad, batch_pad, lat_flat, n_batches):
    npad = pos_pad.shape[0]
    nt = npad // NW
    lat_words = lat_flat.shape[0]
    mesh = plsc.VectorSubcoreMesh(core_axis_name="c", subcore_axis_name="s")

    @functools.partial(
        pl.kernel,
        mesh=mesh,
        out_type=[
            jax.ShapeDtypeStruct((npad, 16), jnp.float32),
            jax.ShapeDtypeStruct((npad, 4), jnp.float32),
        ],
        scratch_types=[
            pltpu.VMEM((nt, 3), jnp.float32),
            pltpu.VMEM((nt,), jnp.int32),
            pltpu.VMEM((lat_words,), jnp.float32),
            pltpu.VMEM((nt, 16), jnp.float32),
            pltpu.VMEM((nt, 4), jnp.float32),
        ],
    )
    def build(pos_hbm, b_hbm, lat_hbm, t_hbm, p4_hbm, posb, bb, latb, tb, p4b):
        wid = lax.axis_index("s") * NC + lax.axis_index("c")
        base = wid * nt
        pltpu.sync_copy(pos_hbm.at[pl.ds(base, nt)], posb)
        pltpu.sync_copy(b_hbm.at[pl.ds(base, nt)], bb)
        pltpu.sync_copy(lat_hbm, latb)
        viota = lax.iota(jnp.int32, 16)

        def body(blk, carry):
            rows = blk * 16 + viota
            b = bb[pl.ds(blk * 16, 16)]
            b9 = jnp.clip(b, 0, n_batches - 1) * 9
            for k in range(3):
                p = plsc.load_gather(posb, [rows, _full(k)])
                plsc.store_scatter(tb, [rows, _full(k)], p)
                plsc.store_scatter(p4b, [rows, _full(k)], p)
            for mk in range(9):
                lv = plsc.load_gather(latb, [b9 + mk])
                plsc.store_scatter(tb, [rows, _full(3 + mk)], lv)
            return carry

        lax.fori_loop(0, nt // 16, body, 0)
        pltpu.sync_copy(tb, t_hbm.at[pl.ds(base, nt)])
        pltpu.sync_copy(p4b, p4_hbm.at[pl.ds(base, nt)])

    return build(pos_pad, batch_pad, lat_flat)


def _edge_kernel(t_tab, p4_tab, ej2, ei2, shift_pad):
    epad = shift_pad.shape[0]
    ept = epad // NW
    n_chunks = ept // _CHUNK
    cb = _CHUNK // _GB
    mesh = plsc.VectorSubcoreMesh(core_axis_name="c", subcore_axis_name="s")

    @functools.partial(
        pl.kernel,
        mesh=mesh,
        out_type=[
            jax.ShapeDtypeStruct((epad,), jnp.float32),
            jax.ShapeDtypeStruct((epad, 3), jnp.float32),
            jax.ShapeDtypeStruct((epad, 3), jnp.float32),
        ],
        scratch_types=[
            pltpu.VMEM((cb, _GB), jnp.int32),
            pltpu.VMEM((cb, _GB), jnp.int32),
            pltpu.VMEM((_CHUNK, 3), jnp.float32),
            pltpu.VMEM((_CHUNK, 16), jnp.float32),
            pltpu.VMEM((_CHUNK, 4), jnp.float32),
            pltpu.VMEM((_CHUNK,), jnp.float32),
            pltpu.VMEM((_CHUNK, 3), jnp.float32),
            pltpu.VMEM((_CHUNK, 3), jnp.float32),
            pltpu.SemaphoreType.DMA,
            pltpu.SemaphoreType.DMA,
        ],
    )
    def edges(t_hbm, p4_hbm, ej_hbm, ei_hbm, sh_hbm,
              dist_hbm, vec_hbm, dir_hbm,
              jidx, iidx, shb, irows, jrows, distb, vecb, dirb,
              sem_i, sem_j):
        wid = lax.axis_index("s") * NC + lax.axis_index("c")
        tbase = wid * ept
        viota = lax.iota(jnp.int32, 16)

        def chunk_body(c, carry):
            g = tbase + c * _CHUNK
            gr = g // _GB
            pltpu.sync_copy(ej_hbm.at[pl.ds(gr, cb)], jidx)
            pltpu.sync_copy(ei_hbm.at[pl.ds(gr, cb)], iidx)
            pltpu.sync_copy(sh_hbm.at[pl.ds(g, _CHUNK)], shb)
            copies = []
            for k in range(cb):
                copies.append(pltpu.async_copy(
                    t_hbm.at[iidx.at[k]],
                    irows.at[pl.ds(k * _GB, _GB)], sem_i))
            for k in range(cb):
                copies.append(pltpu.async_copy(
                    p4_hbm.at[jidx.at[k]],
                    jrows.at[pl.ds(k * _GB, _GB)], sem_j))
            for cp in copies:
                cp.wait()

            def blk(bi, carry2):
                rows = bi * 16 + viota
                s0 = plsc.load_gather(shb, [rows, _full(0)])
                s1 = plsc.load_gather(shb, [rows, _full(1)])
                s2 = plsc.load_gather(shb, [rows, _full(2)])
                v = []
                for k in range(3):
                    pj = plsc.load_gather(jrows, [rows, _full(k)])
                    pi = plsc.load_gather(irows, [rows, _full(k)])
                    l0 = plsc.load_gather(irows, [rows, _full(3 + k)])
                    l1 = plsc.load_gather(irows, [rows, _full(6 + k)])
                    l2 = plsc.load_gather(irows, [rows, _full(9 + k)])
                    v.append(pj - pi + s0 * l0 + s1 * l1 + s2 * l2)
                d2 = v[0] * v[0] + v[1] * v[1] + v[2] * v[2]
                y = _rsqrt(d2)
                distb[pl.ds(bi * 16, 16)] = d2 * y
                for k in range(3):
                    plsc.store_scatter(vecb, [rows, _full(k)], v[k])
                    plsc.store_scatter(dirb, [rows, _full(k)], v[k] * y)
                return carry2

            lax.fori_loop(0, _CHUNK // 16, blk, 0)
            pltpu.sync_copy(distb, dist_hbm.at[pl.ds(g, _CHUNK)])
            pltpu.sync_copy(vecb, vec_hbm.at[pl.ds(g, _CHUNK)])
            pltpu.sync_copy(dirb, dir_hbm.at[pl.ds(g, _CHUNK)])
            return carry

        lax.fori_loop(0, n_chunks, chunk_body, 0)

    return edges(t_tab, p4_tab, ej2, ei2, shift_pad)


def kernel(pos, edge_shift, lattice, edge_index, batch_idx):
    n = pos.shape[0]
    e = edge_shift.shape[0]
    b = lattice.shape[0]

    nt = -(-n // (NW * 16)) * 16
    npad = nt * NW
    pos_pad = jnp.concatenate(
        [pos, jnp.zeros((npad - n, 3), pos.dtype)]) if npad != n else pos
    batch_pad = jnp.concatenate(
        [batch_idx, jnp.zeros((npad - n,), batch_idx.dtype)]) if npad != n else batch_idx
    lat_flat = lattice.reshape(b * 9)

    t_tab, p4_tab = _build_tables(pos_pad, batch_pad, lat_flat, b)

    step = NW * _CHUNK
    epad = -(-e // step) * step
    ej = edge_index[0]
    ei = edge_index[1]
    if epad != e:
        zi = jnp.zeros((epad - e,), jnp.int32)
        ej = jnp.concatenate([ej, zi])
        ei = jnp.concatenate([ei, zi])
        shift_pad = jnp.concatenate(
            [edge_shift, jnp.zeros((epad - e, 3), edge_shift.dtype)])
    else:
        shift_pad = edge_shift
    ej2 = ej.reshape(epad // _GB, _GB)
    ei2 = ei.reshape(epad // _GB, _GB)

    dist, vec, dirn = _edge_kernel(t_tab, p4_tab, ej2, ei2, shift_pad)
    if epad != e:
        dist, vec, dirn = dist[:e], vec[:e], dirn[:e]
    return dist, vec, dirn


# speedup vs baseline: 8.7652x; 8.7652x over previous
import functools

import jax
import jax.numpy as jnp
from jax import lax
from jax.experimental import pallas as pl
from jax.experimental.pallas import tpu as pltpu
from jax.experimental.pallas import tpu_sc as plsc

NC = 2
NS = 16
NW = NC * NS
LANES = 16

_CHUNK = 1024
_GB = 128


def _rsqrt(x):
    xi = plsc.bitcast(x, jnp.int32)
    y = plsc.bitcast(jnp.int32(0x5F3759DF) - (xi >> 1), jnp.float32)
    for _ in range(3):
        y = y * (jnp.float32(1.5) - jnp.float32(0.5) * x * y * y)
    return y


def _full(v):
    return jnp.full((LANES,), v, jnp.int32)


def _build_tables(pos_pad, batch_pad, lat_flat, n_batches):
    npad = pos_pad.shape[0]
    nt = npad // NW
    lat_words = lat_flat.shape[0]
    mesh = plsc.VectorSubcoreMesh(core_axis_name="c", subcore_axis_name="s")

    @functools.partial(
        pl.kernel,
        mesh=mesh,
        compiler_params=pltpu.CompilerParams(needs_layout_passes=False, use_tc_tiling_on_sc=False),
        out_type=[
            jax.ShapeDtypeStruct((npad, 16), jnp.float32),
        ],
        scratch_types=[
            pltpu.VMEM((nt, 3), jnp.float32),
            pltpu.VMEM((nt,), jnp.int32),
            pltpu.VMEM((lat_words,), jnp.float32),
            pltpu.VMEM((nt, 16), jnp.float32),
        ],
    )
    def build(pos_hbm, b_hbm, lat_hbm, t_hbm, posb, bb, latb, tb):
        wid = lax.axis_index("s") * NC + lax.axis_index("c")
        base = wid * nt
        pltpu.sync_copy(pos_hbm.at[pl.ds(base, nt)], posb)
        pltpu.sync_copy(b_hbm.at[pl.ds(base, nt)], bb)
        pltpu.sync_copy(lat_hbm, latb)
        viota = lax.iota(jnp.int32, 16)

        def body(blk, carry):
            rows = blk * 16 + viota
            b = bb[pl.ds(blk * 16, 16)]
            b9 = jnp.clip(b, 0, n_batches - 1) * 9
            for k in range(3):
                p = plsc.load_gather(posb, [rows, _full(k)])
                plsc.store_scatter(tb, [rows, _full(k)], p)
            for mk in range(9):
                lv = plsc.load_gather(latb, [b9 + mk])
                plsc.store_scatter(tb, [rows, _full(3 + mk)], lv)
            return carry

        lax.fori_loop(0, nt // 16, body, 0)
        pltpu.sync_copy(tb, t_hbm.at[pl.ds(base, nt)])

    return build(pos_pad, batch_pad, lat_flat)


def _edge_kernel(t_tab, ej2, ei2, shift_pad):
    epad = shift_pad.shape[0]
    ept = epad // NW
    n_chunks = ept // _CHUNK
    cb = _CHUNK // _GB
    mesh = plsc.VectorSubcoreMesh(core_axis_name="c", subcore_axis_name="s")

    @functools.partial(
        pl.kernel,
        mesh=mesh,
        compiler_params=pltpu.CompilerParams(needs_layout_passes=False, use_tc_tiling_on_sc=False),
        out_type=[
            jax.ShapeDtypeStruct((epad,), jnp.float32),
            jax.ShapeDtypeStruct((epad, 3), jnp.float32),
            jax.ShapeDtypeStruct((epad, 3), jnp.float32),
        ],
        scratch_types=[
            pltpu.VMEM((cb, _GB), jnp.int32),
            pltpu.VMEM((cb, _GB), jnp.int32),
            pltpu.VMEM((_CHUNK, 3), jnp.float32),
            pltpu.VMEM((cb, _GB, 16), jnp.float32),
            pltpu.VMEM((cb, _GB, 16), jnp.float32),
            pltpu.VMEM((_CHUNK,), jnp.float32),
            pltpu.VMEM((_CHUNK, 3), jnp.float32),
            pltpu.VMEM((_CHUNK, 3), jnp.float32),
            pltpu.SemaphoreType.DMA,
            pltpu.SemaphoreType.DMA,
        ],
    )
    def edges(t_hbm, ej_hbm, ei_hbm, sh_hbm,
              dist_hbm, vec_hbm, dir_hbm,
              jidx, iidx, shb, irows, jrows, distb, vecb, dirb,
              sem_i, sem_j):
        wid = lax.axis_index("s") * NC + lax.axis_index("c")
        tbase = wid * ept
        viota = lax.iota(jnp.int32, 16)

        def chunk_body(c, carry):
            g = tbase + c * _CHUNK
            gr = g // _GB
            pltpu.sync_copy(ej_hbm.at[pl.ds(gr, cb)], jidx)
            pltpu.sync_copy(ei_hbm.at[pl.ds(gr, cb)], iidx)
            pltpu.sync_copy(sh_hbm.at[pl.ds(g, _CHUNK)], shb)
            copies = []
            for k in range(cb):
                copies.append(pltpu.async_copy(
                    t_hbm.at[iidx.at[k]], irows.at[k], sem_i))
            for k in range(cb):
                copies.append(pltpu.async_copy(
                    t_hbm.at[jidx.at[k]], jrows.at[k], sem_j))
            for cp in copies:
                cp.wait()

            def blk(bi, carry2):
                rows = bi * 16 + viota
                q = rows >> 7
                w = rows & 127
                s0 = plsc.load_gather(shb, [rows, _full(0)])
                s1 = plsc.load_gather(shb, [rows, _full(1)])
                s2 = plsc.load_gather(shb, [rows, _full(2)])
                v = []
                for k in range(3):
                    pj = plsc.load_gather(jrows, [q, w, _full(k)])
                    pi = plsc.load_gather(irows, [q, w, _full(k)])
                    l0 = plsc.load_gather(irows, [q, w, _full(3 + k)])
                    l1 = plsc.load_gather(irows, [q, w, _full(6 + k)])
                    l2 = plsc.load_gather(irows, [q, w, _full(9 + k)])
                    v.append(pj - pi + s0 * l0 + s1 * l1 + s2 * l2)
                d2 = v[0] * v[0] + v[1] * v[1] + v[2] * v[2]
                y = _rsqrt(d2)
                distb[pl.ds(bi * 16, 16)] = d2 * y
                for k in range(3):
                    plsc.store_scatter(vecb, [rows, _full(k)], v[k])
                    plsc.store_scatter(dirb, [rows, _full(k)], v[k] * y)
                return carry2

            lax.fori_loop(0, _CHUNK // 16, blk, 0)
            pltpu.sync_copy(distb, dist_hbm.at[pl.ds(g, _CHUNK)])
            pltpu.sync_copy(vecb, vec_hbm.at[pl.ds(g, _CHUNK)])
            pltpu.sync_copy(dirb, dir_hbm.at[pl.ds(g, _CHUNK)])
            return carry

        lax.fori_loop(0, n_chunks, chunk_body, 0)

    return edges(t_tab, ej2, ei2, shift_pad)


def kernel(pos, edge_shift, lattice, edge_index, batch_idx):
    n = pos.shape[0]
    e = edge_shift.shape[0]
    b = lattice.shape[0]

    nt = -(-n // (NW * 16)) * 16
    npad = nt * NW
    pos_pad = jnp.concatenate(
        [pos, jnp.zeros((npad - n, 3), pos.dtype)]) if npad != n else pos
    batch_pad = jnp.concatenate(
        [batch_idx, jnp.zeros((npad - n,), batch_idx.dtype)]) if npad != n else batch_idx
    lat_flat = lattice.reshape(b * 9)

    t_tab, = _build_tables(pos_pad, batch_pad, lat_flat, b)

    step = NW * _CHUNK
    epad = -(-e // step) * step
    ej = edge_index[0]
    ei = edge_index[1]
    if epad != e:
        zi = jnp.zeros((epad - e,), jnp.int32)
        ej = jnp.concatenate([ej, zi])
        ei = jnp.concatenate([ei, zi])
        shift_pad = jnp.concatenate(
            [edge_shift, jnp.zeros((epad - e, 3), edge_shift.dtype)])
    else:
        shift_pad = edge_shift
    ej2 = ej.reshape(epad // _GB, _GB)
    ei2 = ei.reshape(epad // _GB, _GB)

    dist, vec, dirn = _edge_kernel(t_tab, ej2, ei2, shift_pad)
    if epad != e:
        dist, vec, dirn = dist[:e], vec[:e], dirn[:e]
    return dist, vec, dirn


# speedup vs baseline: 10.9263x; 1.2466x over previous
import functools

import jax
import jax.numpy as jnp
from jax import lax
from jax.experimental import pallas as pl
from jax.experimental.pallas import tpu as pltpu
from jax.experimental.pallas import tpu_sc as plsc

NC = 2
NS = 16
NW = NC * NS
LANES = 16

_CHUNK = 1024
_GB = 128

_PARAMS = pltpu.CompilerParams(
    needs_layout_passes=False, use_tc_tiling_on_sc=False)


def _rsqrt(x):
    xi = plsc.bitcast(x, jnp.int32)
    y = plsc.bitcast(jnp.int32(0x5F3759DF) - (xi >> 1), jnp.float32)
    for _ in range(3):
        y = y * (jnp.float32(1.5) - jnp.float32(0.5) * x * y * y)
    return y


def _full(v):
    return jnp.full((LANES,), v, jnp.int32)


def _build_table(pos, batch_idx, lat_flat, n_batches):
    n = pos.shape[0]
    nt = -(-n // (NW * 16)) * 16
    lat_words = lat_flat.shape[0]
    mesh = plsc.VectorSubcoreMesh(core_axis_name="c", subcore_axis_name="s")

    @functools.partial(
        pl.kernel,
        mesh=mesh,
        compiler_params=_PARAMS,
        out_type=[jax.ShapeDtypeStruct((n, 16), jnp.float32)],
        scratch_types=[
            pltpu.VMEM((nt, 3), jnp.float32),
            pltpu.VMEM((nt,), jnp.int32),
            pltpu.VMEM((lat_words,), jnp.float32),
            pltpu.VMEM((nt, 16), jnp.float32),
        ],
    )
    def build(pos_hbm, b_hbm, lat_hbm, t_hbm, posb, bb, latb, tb):
        wid = lax.axis_index("s") * NC + lax.axis_index("c")
        base = jnp.minimum(wid * nt, n - nt)
        pltpu.sync_copy(pos_hbm.at[pl.ds(base, nt)], posb)
        pltpu.sync_copy(b_hbm.at[pl.ds(base, nt)], bb)
        pltpu.sync_copy(lat_hbm, latb)
        viota = lax.iota(jnp.int32, 16)

        def body(blk, carry):
            rows = blk * 16 + viota
            b = bb[pl.ds(blk * 16, 16)]
            b9 = jnp.clip(b, 0, n_batches - 1) * 9
            for k in range(3):
                p = plsc.load_gather(posb, [rows, _full(k)])
                plsc.store_scatter(tb, [rows, _full(k)], p)
            for mk in range(9):
                lv = plsc.load_gather(latb, [b9 + mk])
                plsc.store_scatter(tb, [rows, _full(3 + mk)], lv)
            return carry

        lax.fori_loop(0, nt // 16, body, 0)
        pltpu.sync_copy(tb, t_hbm.at[pl.ds(base, nt)])

    return build(pos, batch_idx, lat_flat)


def _edge_kernel(t_tab, edge_index, shift):
    e = shift.shape[0]
    ept = e // NW
    n_chunks = -(-ept // _CHUNK)
    cb = _CHUNK // _GB
    mesh = plsc.VectorSubcoreMesh(core_axis_name="c", subcore_axis_name="s")

    @functools.partial(
        pl.kernel,
        mesh=mesh,
        compiler_params=_PARAMS,
        out_type=[
            jax.ShapeDtypeStruct((e,), jnp.float32),
            jax.ShapeDtypeStruct((e, 3), jnp.float32),
            jax.ShapeDtypeStruct((e, 3), jnp.float32),
        ],
        scratch_types=[
            pltpu.VMEM((_CHUNK,), jnp.int32),
            pltpu.VMEM((_CHUNK,), jnp.int32),
            pltpu.VMEM((_CHUNK, 3), jnp.float32),
            pltpu.VMEM((cb, _GB, 16), jnp.float32),
            pltpu.VMEM((cb, _GB, 16), jnp.float32),
            pltpu.VMEM((_CHUNK,), jnp.float32),
            pltpu.VMEM((_CHUNK, 3), jnp.float32),
            pltpu.VMEM((_CHUNK, 3), jnp.float32),
            pltpu.SemaphoreType.DMA,
            pltpu.SemaphoreType.DMA,
        ],
    )
    def edges(t_hbm, eidx_hbm, sh_hbm,
              dist_hbm, vec_hbm, dir_hbm,
              jidx, iidx, shb, irows, jrows, distb, vecb, dirb,
              sem_i, sem_j):
        wid = lax.axis_index("s") * NC + lax.axis_index("c")
        tbase = wid * ept
        viota = lax.iota(jnp.int32, 16)

        def chunk_body(c, carry):
            g = tbase + jnp.minimum(c * _CHUNK, ept - _CHUNK)
            pltpu.sync_copy(eidx_hbm.at[0, pl.ds(g, _CHUNK)], jidx)
            pltpu.sync_copy(eidx_hbm.at[1, pl.ds(g, _CHUNK)], iidx)
            pltpu.sync_copy(sh_hbm.at[pl.ds(g, _CHUNK)], shb)
            copies = []
            for k in range(cb):
                copies.append(pltpu.async_copy(
                    t_hbm.at[iidx.at[pl.ds(k * _GB, _GB)]], irows.at[k],
                    sem_i))
            for k in range(cb):
                copies.append(pltpu.async_copy(
                    t_hbm.at[jidx.at[pl.ds(k * _GB, _GB)]], jrows.at[k],
                    sem_j))
            for cp in copies:
                cp.wait()

            def blk(bi, carry2):
                rows = bi * 16 + viota
                q = rows >> 7
                w = rows & 127
                s0 = plsc.load_gather(shb, [rows, _full(0)])
                s1 = plsc.load_gather(shb, [rows, _full(1)])
                s2 = plsc.load_gather(shb, [rows, _full(2)])
                v = []
                for k in range(3):
                    pj = plsc.load_gather(jrows, [q, w, _full(k)])
                    pi = plsc.load_gather(irows, [q, w, _full(k)])
                    l0 = plsc.load_gather(irows, [q, w, _full(3 + k)])
                    l1 = plsc.load_gather(irows, [q, w, _full(6 + k)])
                    l2 = plsc.load_gather(irows, [q, w, _full(9 + k)])
                    v.append(pj - pi + s0 * l0 + s1 * l1 + s2 * l2)
                d2 = v[0] * v[0] + v[1] * v[1] + v[2] * v[2]
                y = _rsqrt(d2)
                distb[pl.ds(bi * 16, 16)] = d2 * y
                for k in range(3):
                    plsc.store_scatter(vecb, [rows, _full(k)], v[k])
                    plsc.store_scatter(dirb, [rows, _full(k)], v[k] * y)
                return carry2

            lax.fori_loop(0, _CHUNK // 16, blk, 0)
            pltpu.sync_copy(distb, dist_hbm.at[pl.ds(g, _CHUNK)])
            pltpu.sync_copy(vecb, vec_hbm.at[pl.ds(g, _CHUNK)])
            pltpu.sync_copy(dirb, dir_hbm.at[pl.ds(g, _CHUNK)])
            return carry

        lax.fori_loop(0, n_chunks, chunk_body, 0)

    return edges(t_tab, edge_index, shift)


def kernel(pos, edge_shift, lattice, edge_index, batch_idx):
    n = pos.shape[0]
    e = edge_shift.shape[0]
    b = lattice.shape[0]
    lat_flat = lattice.reshape(b * 9)

    nt = -(-n // (NW * 16)) * 16
    if n < nt or (n - nt) % 16 != 0:
        npad = nt * NW
        pos = jnp.concatenate([pos, jnp.zeros((npad - n, 3), pos.dtype)])
        batch_idx = jnp.concatenate(
            [batch_idx, jnp.zeros((npad - n,), batch_idx.dtype)])
    t_tab, = _build_table(pos, batch_idx, lat_flat, b)

    epad = e
    ept = e // NW
    if e % NW != 0 or ept % 16 != 0 or ept < _CHUNK:
        step = NW * _CHUNK
        epad = -(-e // step) * step
        pad = epad - e
        edge_index = jnp.concatenate(
            [edge_index, jnp.zeros((2, pad), edge_index.dtype)], axis=1)
        edge_shift = jnp.concatenate(
            [edge_shift, jnp.zeros((pad, 3), edge_shift.dtype)])

    dist, vec, dirn = _edge_kernel(t_tab, edge_index, edge_shift)
    if epad != e:
        dist, vec, dirn = dist[:e], vec[:e], dirn[:e]
    return dist, vec, dirn
